# Initial kernel scaffold; baseline (speedup 1.0000x reference)
import jax
import jax.numpy as jnp
from jax.experimental import pallas as pl


def kernel(x, edge_index, edge_weight, W1, b1, W2, b2):
    raise NotImplementedError("write your pallas kernel here")


# speedup vs baseline: 17.7677x; 17.7677x over previous
import functools

import jax
import jax.numpy as jnp
from jax import lax
from jax.experimental import pallas as pl
from jax.experimental.pallas import tpu as pltpu
from jax.experimental.pallas import tpu_sc as plsc

_N = 10000
_NP = 10240
_H1 = 16

_NC = 2
_NS = 16
_NW = _NC * _NS
_G = 128
_GPC = 8
_CHUNK = _G * _GPC
_RPT = _NP // _NS


def _tc_linear1(x, W1):
    n = x.shape[0]

    def body(x_ref, w_ref, o_ref):
        o_ref[pl.ds(0, n), :] = jnp.dot(x_ref[...], w_ref[...],
                                        preferred_element_type=jnp.float32)
        o_ref[pl.ds(n, _NP - n), :] = jnp.zeros((_NP - n, _H1), jnp.float32)

    return pl.pallas_call(
        body,
        out_shape=jax.ShapeDtypeStruct((_NP, W1.shape[1]), jnp.float32),
    )(x, W1)


def _tc_mid(p, b1, W2p):
    def body(p_ref, b_ref, w_ref, o_ref):
        h = jnp.maximum(p_ref[0] + p_ref[1] + b_ref[...], 0.0)
        o_ref[...] = jnp.dot(h, w_ref[...], preferred_element_type=jnp.float32)

    return pl.pallas_call(
        body,
        out_shape=jax.ShapeDtypeStruct((p.shape[1], W2p.shape[1]), jnp.float32),
    )(p, b1, W2p)


def _tc_final(p, b2p):
    def body(p_ref, b_ref, o_ref):
        o_ref[...] = p_ref[0] + p_ref[1] + b_ref[...]

    return pl.pallas_call(
        body,
        out_shape=jax.ShapeDtypeStruct((p.shape[1], p.shape[2]), jnp.float32),
    )(p, b2p)


def _make_agg(n_chunks):
    mesh = plsc.VectorSubcoreMesh(core_axis_name="c", subcore_axis_name="s")

    @functools.partial(
        pl.kernel,
        out_type=jax.ShapeDtypeStruct((_NC, _NP, _H1), jnp.float32),
        mesh=mesh,
        compiler_params=pltpu.CompilerParams(use_tc_tiling_on_sc=False),
        scratch_types=[
            pltpu.VMEM_SHARED((_NP, _H1), jnp.float32),
            pltpu.VMEM_SHARED((_NP, _H1), jnp.float32),
            pltpu.VMEM((_GPC, _G), jnp.int32),
            pltpu.VMEM((_GPC, _G), jnp.int32),
            pltpu.VMEM((_GPC, _G), jnp.float32),
            pltpu.VMEM((_GPC, _G, _H1), jnp.float32),
        ],
    )
    def agg(table_hbm, src_hbm, dst_hbm, ew_hbm, zero_hbm, out_hbm,
            table_sh, acc_sh, src_v, dst_v, w_v, rows_v):
        c = lax.axis_index("c")
        s = lax.axis_index("s")
        wid = s * _NC + c
        r0 = s * _RPT
        pltpu.sync_copy(table_hbm.at[pl.ds(r0, _RPT)], table_sh.at[pl.ds(r0, _RPT)])
        pltpu.sync_copy(zero_hbm.at[pl.ds(r0, _RPT)], acc_sh.at[pl.ds(r0, _RPT)])
        plsc.subcore_barrier()

        def chunk_body(i, carry):
            pltpu.sync_copy(src_hbm.at[wid, i], src_v)
            pltpu.sync_copy(dst_hbm.at[wid, i], dst_v)
            pltpu.sync_copy(ew_hbm.at[wid, i], w_v)
            for a in range(_GPC):
                pltpu.sync_copy(table_sh.at[src_v.at[a]], rows_v.at[a])
            for a in range(_GPC):
                def scale(b, cc, a=a):
                    j0 = b * 16
                    wvec = w_v[a, pl.ds(j0, 16)]
                    for r in range(16):
                        rows_v[a, j0 + r, :] = rows_v[a, j0 + r, :] * wvec[r]
                    return cc
                lax.fori_loop(0, _G // 16, scale, 0)
            for a in range(_GPC):
                pltpu.sync_copy(rows_v.at[a], acc_sh.at[dst_v.at[a]], add=True)
            return carry

        lax.fori_loop(0, n_chunks, chunk_body, 0)
        plsc.subcore_barrier()
        pltpu.sync_copy(acc_sh.at[pl.ds(r0, _RPT)], out_hbm.at[c, pl.ds(r0, _RPT)])

    return agg


def kernel(x, edge_index, edge_weight, W1, b1, W2, b2):
    E = edge_weight.shape[0]
    H2 = W2.shape[1]
    per_tile = -(-E // (_NW * _CHUNK))
    EP = _NW * per_tile * _CHUNK
    pad = EP - E
    src = edge_index[1]
    dst = edge_index[0]
    ew = edge_weight
    if pad:
        fill = jnp.arange(pad, dtype=jnp.int32) % _N
        src = jnp.concatenate([src, fill])
        dst = jnp.concatenate([dst, fill])
        ew = jnp.concatenate([ew, jnp.zeros((pad,), jnp.float32)])
    srcp = src.reshape(_NW, per_tile, _GPC, _G)
    dstp = dst.reshape(_NW, per_tile, _GPC, _G)
    ewp = ew.reshape(_NW, per_tile, _GPC, _G)
    zeros = jnp.zeros((_NP, _H1), jnp.float32)

    agg = _make_agg(per_tile)

    xw = _tc_linear1(x, W1)
    p1 = agg(xw, srcp, dstp, ewp, zeros)

    W2p = jnp.zeros((_H1, _H1), jnp.float32).at[:, :H2].set(W2)
    hw = _tc_mid(p1, b1.reshape(1, _H1), W2p)
    p2 = agg(hw, srcp, dstp, ewp, zeros)

    b2p = jnp.zeros((1, _H1), jnp.float32).at[0, :H2].set(b2)
    out16 = _tc_final(p2, b2p)
    return out16[:_N, :H2]


# speedup vs baseline: 27.1320x; 1.5270x over previous
import functools

import jax
import jax.numpy as jnp
from jax import lax
from jax.experimental import pallas as pl
from jax.experimental.pallas import tpu as pltpu
from jax.experimental.pallas import tpu_sc as plsc

_N = 10000
_NP = 10240
_H1 = 16

_NC = 2
_NS = 16
_NW = _NC * _NS
_G = 128
_GPC = 8
_CHUNK = _G * _GPC
_RPT = _NP // _NS


def _tc_linear1(x, W1):
    n = x.shape[0]

    def body(x_ref, w_ref, o_ref):
        o_ref[pl.ds(0, n), :] = jnp.dot(x_ref[...], w_ref[...],
                                        preferred_element_type=jnp.float32)
        o_ref[pl.ds(n, _NP - n), :] = jnp.zeros((_NP - n, _H1), jnp.float32)

    return pl.pallas_call(
        body,
        out_shape=jax.ShapeDtypeStruct((_NP, W1.shape[1]), jnp.float32),
    )(x, W1)


def _tc_mid(p, b1, W2p):
    def body(p_ref, b_ref, w_ref, o_ref):
        h = jnp.maximum(p_ref[0] + p_ref[1] + b_ref[...], 0.0)
        o_ref[...] = jnp.dot(h, w_ref[...], preferred_element_type=jnp.float32)

    return pl.pallas_call(
        body,
        out_shape=jax.ShapeDtypeStruct((p.shape[1], W2p.shape[1]), jnp.float32),
    )(p, b1, W2p)


def _tc_final(p, b2p):
    def body(p_ref, b_ref, o_ref):
        o_ref[...] = p_ref[0] + p_ref[1] + b_ref[...]

    return pl.pallas_call(
        body,
        out_shape=jax.ShapeDtypeStruct((p.shape[1], p.shape[2]), jnp.float32),
    )(p, b2p)


def _make_agg(n_chunks):
    mesh = plsc.VectorSubcoreMesh(core_axis_name="c", subcore_axis_name="s")

    @functools.partial(
        pl.kernel,
        out_type=jax.ShapeDtypeStruct((_NC, _NP, _H1), jnp.float32),
        mesh=mesh,
        compiler_params=pltpu.CompilerParams(use_tc_tiling_on_sc=False),
        scratch_types=[
            pltpu.VMEM_SHARED((_NP, _H1), jnp.float32),
            pltpu.VMEM_SHARED((_NP, _H1), jnp.float32),
            pltpu.VMEM((2, _GPC, _G), jnp.int32),
            pltpu.VMEM((2, _GPC, _G), jnp.int32),
            pltpu.VMEM((2, _GPC, _G), jnp.float32),
            pltpu.VMEM((_GPC, _G, _H1), jnp.float32),
            pltpu.SemaphoreType.DMA((2,)),
            pltpu.SemaphoreType.DMA((_GPC,)),
            pltpu.SemaphoreType.DMA,
            pltpu.SemaphoreType.DMA,
        ],
    )
    def agg(table_hbm, src_hbm, dst_hbm, ew_hbm, zero_hbm, out_hbm,
            table_sh, acc_sh, src_v, dst_v, w_v, rows_v,
            esem, gsem, ssem, stsem):
        c = lax.axis_index("c")
        s = lax.axis_index("s")
        wid = s * _NC + c
        r0 = s * _RPT
        st1 = pltpu.async_copy(table_hbm.at[pl.ds(r0, _RPT)],
                               table_sh.at[pl.ds(r0, _RPT)], stsem)
        st2 = pltpu.async_copy(zero_hbm.at[pl.ds(r0, _RPT)],
                               acc_sh.at[pl.ds(r0, _RPT)], stsem)

        def fetch_edges(i, p):
            return (pltpu.async_copy(src_hbm.at[wid, i], src_v.at[p], esem.at[p]),
                    pltpu.async_copy(dst_hbm.at[wid, i], dst_v.at[p], esem.at[p]),
                    pltpu.async_copy(ew_hbm.at[wid, i], w_v.at[p], esem.at[p]))

        for d in fetch_edges(0, 0):
            pass
        st1.wait()
        st2.wait()
        plsc.subcore_barrier()

        def chunk_body(i, carry):
            p = lax.rem(i, 2)
            pltpu.make_async_copy(src_hbm.at[wid, i], src_v.at[p], esem.at[p]).wait()
            pltpu.make_async_copy(dst_hbm.at[wid, i], dst_v.at[p], esem.at[p]).wait()
            pltpu.make_async_copy(ew_hbm.at[wid, i], w_v.at[p], esem.at[p]).wait()

            @pl.when(i + 1 < n_chunks)
            def _prefetch():
                fetch_edges(i + 1, 1 - p)

            gathers = [
                pltpu.async_copy(table_sh.at[src_v.at[p, a]], rows_v.at[a],
                                 gsem.at[a])
                for a in range(_GPC)
            ]
            scatters = []
            for a in range(_GPC):
                gathers[a].wait()

                def scale(b, cc, a=a):
                    j0 = b * 16
                    wvec = w_v[p, a, pl.ds(j0, 16)]
                    for r in range(16):
                        rows_v[a, j0 + r, :] = rows_v[a, j0 + r, :] * wvec[r]
                    return cc

                lax.fori_loop(0, _G // 16, scale, 0)
                scatters.append(
                    pltpu.async_copy(rows_v.at[a], acc_sh.at[dst_v.at[p, a]],
                                     ssem, add=True))
            for d in scatters:
                d.wait()
            return carry

        lax.fori_loop(0, n_chunks, chunk_body, 0)
        plsc.subcore_barrier()
        pltpu.sync_copy(acc_sh.at[pl.ds(r0, _RPT)], out_hbm.at[c, pl.ds(r0, _RPT)])

    return agg


def kernel(x, edge_index, edge_weight, W1, b1, W2, b2):
    E = edge_weight.shape[0]
    H2 = W2.shape[1]
    per_tile = -(-E // (_NW * _CHUNK))
    EP = _NW * per_tile * _CHUNK
    pad = EP - E
    src = edge_index[1]
    dst = edge_index[0]
    ew = edge_weight
    if pad:
        fill = jnp.arange(pad, dtype=jnp.int32) % _N
        src = jnp.concatenate([src, fill])
        dst = jnp.concatenate([dst, fill])
        ew = jnp.concatenate([ew, jnp.zeros((pad,), jnp.float32)])
    srcp = src.reshape(_NW, per_tile, _GPC, _G)
    dstp = dst.reshape(_NW, per_tile, _GPC, _G)
    ewp = ew.reshape(_NW, per_tile, _GPC, _G)
    zeros = jnp.zeros((_NP, _H1), jnp.float32)

    agg = _make_agg(per_tile)

    xw = _tc_linear1(x, W1)
    p1 = agg(xw, srcp, dstp, ewp, zeros)

    W2p = jnp.zeros((_H1, _H1), jnp.float32).at[:, :H2].set(W2)
    hw = _tc_mid(p1, b1.reshape(1, _H1), W2p)
    p2 = agg(hw, srcp, dstp, ewp, zeros)

    b2p = jnp.zeros((1, _H1), jnp.float32).at[0, :H2].set(b2)
    out16 = _tc_final(p2, b2p)
    return out16[:_N, :H2]
